# Initial kernel scaffold; baseline (speedup 1.0000x reference)
import jax
import jax.numpy as jnp
from jax.experimental import pallas as pl


def kernel(x, edge_attr, edge_index, params):
    raise NotImplementedError("write your pallas kernel here")


# speedup vs baseline: 2.0723x; 2.0723x over previous
import functools

import jax
import jax.numpy as jnp
from jax import lax
from jax.experimental import pallas as pl
from jax.experimental.pallas import tpu as pltpu
from jax.experimental.pallas import tpu_sc as plsc

N_NODES = 10000
N_EDGES = 320000
D = 128

NC = 2
NS = 16
NW = NC * NS

CHUNK = 128
G_CHUNKS = (2 * N_EDGES) // CHUNK
S_CHUNKS = N_EDGES // CHUNK
ZROW = 80
ZCHUNKS = N_NODES // ZROW

_mesh = plsc.VectorSubcoreMesh(core_axis_name="c", subcore_axis_name="s")


def _relu(v):
    return jnp.maximum(v, 0.0)


def _dot(a, b):
    return jnp.dot(a, b, preferred_element_type=jnp.float32)


def _gather_body(table, idx, out, idx_v, buf_v, sem):
    c = lax.axis_index("c")
    s = lax.axis_index("s")
    wid = s * NC + c

    @pl.loop(wid, G_CHUNKS, step=NW)
    def _(k):
        pltpu.sync_copy(idx.at[k], idx_v)
        pltpu.async_copy(table.at[idx_v], buf_v, sem).wait()
        pltpu.sync_copy(buf_v, out.at[pl.ds(k * CHUNK, CHUNK)])


_sc_gather = pl.kernel(
    _gather_body,
    out_type=jax.ShapeDtypeStruct((2 * N_EDGES, D), jnp.float32),
    mesh=_mesh,
    scratch_types=[
        pltpu.VMEM((CHUNK,), jnp.int32),
        pltpu.VMEM((CHUNK, D), jnp.float32),
        pltpu.SemaphoreType.DMA,
    ],
)


def _scatter_body(msg, idx, zeros, out0, out1, shared, idx_v, mbuf, sem):
    c = lax.axis_index("c")
    s = lax.axis_index("s")
    wid = s * NC + c

    @pl.loop(s, ZCHUNKS, step=NS)
    def _(z):
        pltpu.sync_copy(zeros.at[pl.ds(z * ZROW, ZROW)],
                        shared.at[pl.ds(z * ZROW, ZROW)])
    plsc.subcore_barrier()

    @pl.loop(wid, S_CHUNKS, step=NW)
    def _(k):
        pltpu.sync_copy(idx.at[pl.ds(k, 1)], idx_v)
        pltpu.sync_copy(msg.at[pl.ds(k * CHUNK, CHUNK)], mbuf)
        pltpu.sync_copy(mbuf, shared.at[idx_v.at[0]], add=True)

    plsc.subcore_barrier()

    @pl.when(c == 0)
    def _():
        @pl.loop(s, ZCHUNKS, step=NS)
        def _(z):
            pltpu.sync_copy(shared.at[pl.ds(z * ZROW, ZROW)],
                            out0.at[pl.ds(z * ZROW, ZROW)])

    @pl.when(c == 1)
    def _():
        @pl.loop(s, ZCHUNKS, step=NS)
        def _(z):
            pltpu.sync_copy(shared.at[pl.ds(z * ZROW, ZROW)],
                            out1.at[pl.ds(z * ZROW, ZROW)])


_sc_scatter = pl.kernel(
    _scatter_body,
    out_type=(
        jax.ShapeDtypeStruct((N_NODES, D), jnp.float32),
        jax.ShapeDtypeStruct((N_NODES, D), jnp.float32),
    ),
    mesh=_mesh,
    scratch_types=[
        pltpu.VMEM_SHARED((N_NODES, D), jnp.float32),
        pltpu.VMEM((1, CHUNK), jnp.int32),
        pltpu.VMEM((CHUNK, D), jnp.float32),
        pltpu.SemaphoreType.DMA,
    ],
)


N_BLK = 1000


def _node_body(x, w0, b0, w1, b1, w2, b2, o):
    h = _relu(_dot(x[...], w0[...]) + b0[...])
    h = _relu(_dot(h, w1[...]) + b1[...])
    o[...] = _dot(h, w2[...]) + b2[...]


BLK_E = 2000
NBLK_E = N_EDGES // BLK_E


def _full(shape):
    return pl.BlockSpec(shape, lambda c: (0, 0))


def _step1_body(ea, xj, xi, e0w, e0b, e1w, e1b, e2w, e2b, e3w, e3b,
                m0w, m0b, m1w, m1b, n0w, n0b, ef_o, msg_o):
    h = _relu(_dot(ea[...], e0w[...]) + e0b[...])
    h = _relu(_dot(h, e1w[...]) + e1b[...])
    h = _relu(_dot(h, e2w[...]) + e2b[...])
    ef = _dot(h, e3w[...]) + e3b[...]
    xiv = xi[...]
    cat = jnp.concatenate([xiv, xj[...]], axis=1)
    m0 = m0w[...]
    h = _relu(_dot(cat, m0[:256]) + _dot(ef, m0[256:]) + m0b[...])
    ef1 = _relu(_dot(h, m1w[...]) + m1b[...])
    ef_o[...] = ef1
    n0 = n0w[...]
    msg_o[...] = _relu(_dot(xiv, n0[:128]) + _dot(ef1, n0[128:]) + n0b[...])


def _step2_body(ef, xj, xi, m0w, m0b, m1w, m1b, n0w, n0b, ef_o, msg_o):
    xiv = xi[...]
    cat = jnp.concatenate([xiv, xj[...]], axis=1)
    m0 = m0w[...]
    h = _relu(_dot(cat, m0[:256]) + _dot(ef[...], m0[256:]) + m0b[...])
    ef1 = _relu(_dot(h, m1w[...]) + m1b[...])
    ef_o[...] = ef1
    n0 = n0w[...]
    msg_o[...] = _relu(_dot(xiv, n0[:128]) + _dot(ef1, n0[128:]) + n0b[...])


def _step3_body(ef, xj, xi, m0w, m0b, m1w, m1b, c0w, c0b, c1w, c1b,
                c2w, c2b, o):
    cat = jnp.concatenate([xi[...], xj[...]], axis=1)
    m0 = m0w[...]
    h = _relu(_dot(cat, m0[:256]) + _dot(ef[...], m0[256:]) + m0b[...])
    ef1 = _relu(_dot(h, m1w[...]) + m1b[...])
    h = _relu(_dot(ef1, c0w[...]) + c0b[...])
    h = _relu(_dot(h, c1w[...]) + c1b[...])
    o[...] = _dot(h, c2w[...]) + c2b[...]


def _combine_body(a, b, o):
    o[...] = a[...] + b[...]


def _edge_spec(width):
    return pl.BlockSpec((BLK_E, width), lambda c: (c, 0))


def _xj_spec():
    return pl.BlockSpec((BLK_E, D), lambda c: (c, 0))


def _xi_spec():
    return pl.BlockSpec((BLK_E, D), lambda c: (c + NBLK_E, 0))


def kernel(x, edge_attr, edge_index, params):
    p = params

    def wb(name):
        w = p[name + "_W"]
        b = p[name + "_b"].reshape(1, -1)
        return w, b

    ne0w, ne0b = wb("ne0"); ne1w, ne1b = wb("ne1"); ne2w, ne2b = wb("ne2")
    ee0w, ee0b = wb("ee0"); ee1w, ee1b = wb("ee1")
    ee2w, ee2b = wb("ee2"); ee3w, ee3b = wb("ee3")
    me0w, me0b = wb("me0"); me1w, me1b = wb("me1")
    mn0w, mn0b = wb("mn0")
    c0w, c0b = wb("c0"); c1w, c1b = wb("c1"); c2w, c2b = wb("c2")

    idx_all = edge_index.reshape(G_CHUNKS, CHUNK)
    idx_i = edge_index[1].reshape(S_CHUNKS, CHUNK)
    zeros = jnp.zeros((N_NODES, D), jnp.float32)

    nf = pl.pallas_call(
        _node_body,
        grid=(N_NODES // N_BLK,),
        in_specs=[
            pl.BlockSpec((N_BLK, D), lambda c: (c, 0)),
            _full((D, D)), _full((1, D)),
            _full((D, 64)), _full((1, 64)),
            _full((64, D)), _full((1, D)),
        ],
        out_specs=pl.BlockSpec((N_BLK, D), lambda c: (c, 0)),
        out_shape=jax.ShapeDtypeStruct((N_NODES, D), jnp.float32),
    )(x, ne0w, ne0b, ne1w, ne1b, ne2w, ne2b)

    g = _sc_gather(nf, idx_all)
    ef, msg = pl.pallas_call(
        _step1_body,
        grid=(NBLK_E,),
        in_specs=[
            _edge_spec(16), _xj_spec(), _xi_spec(),
            _full((16, 32)), _full((1, 32)),
            _full((32, 64)), _full((1, 64)),
            _full((64, 64)), _full((1, 64)),
            _full((64, 16)), _full((1, 16)),
            _full((272, 64)), _full((1, 64)),
            _full((64, 16)), _full((1, 16)),
            _full((144, D)), _full((1, D)),
        ],
        out_specs=[_edge_spec(16), _edge_spec(D)],
        out_shape=[
            jax.ShapeDtypeStruct((N_EDGES, 16), jnp.float32),
            jax.ShapeDtypeStruct((N_EDGES, D), jnp.float32),
        ],
    )(edge_attr, g, g, ee0w, ee0b, ee1w, ee1b, ee2w, ee2b, ee3w, ee3b,
      me0w, me0b, me1w, me1b, mn0w, mn0b)

    def _segment_sum(m):
        p0, p1 = _sc_scatter(m, idx_i, zeros)
        return pl.pallas_call(
            _combine_body,
            grid=(N_NODES // N_BLK,),
            in_specs=[pl.BlockSpec((N_BLK, D), lambda c: (c, 0))] * 2,
            out_specs=pl.BlockSpec((N_BLK, D), lambda c: (c, 0)),
            out_shape=jax.ShapeDtypeStruct((N_NODES, D), jnp.float32),
        )(p0, p1)

    nf = _segment_sum(msg)

    g = _sc_gather(nf, idx_all)
    ef, msg = pl.pallas_call(
        _step2_body,
        grid=(NBLK_E,),
        in_specs=[
            _edge_spec(16), _xj_spec(), _xi_spec(),
            _full((272, 64)), _full((1, 64)),
            _full((64, 16)), _full((1, 16)),
            _full((144, D)), _full((1, D)),
        ],
        out_specs=[_edge_spec(16), _edge_spec(D)],
        out_shape=[
            jax.ShapeDtypeStruct((N_EDGES, 16), jnp.float32),
            jax.ShapeDtypeStruct((N_EDGES, D), jnp.float32),
        ],
    )(ef, g, g, me0w, me0b, me1w, me1b, mn0w, mn0b)
    nf = _segment_sum(msg)

    g = _sc_gather(nf, idx_all)
    out = pl.pallas_call(
        _step3_body,
        grid=(NBLK_E,),
        in_specs=[
            _edge_spec(16), _xj_spec(), _xi_spec(),
            _full((272, 64)), _full((1, 64)),
            _full((64, 16)), _full((1, 16)),
            _full((16, 64)), _full((1, 64)),
            _full((64, 32)), _full((1, 32)),
            _full((32, 1)), _full((1, 1)),
        ],
        out_specs=_edge_spec(1),
        out_shape=jax.ShapeDtypeStruct((N_EDGES, 1), jnp.float32),
    )(ef, g, g, me0w, me0b, me1w, me1b, c0w, c0b, c1w, c1b, c2w, c2b)
    return out


# speedup vs baseline: 2.6392x; 1.2735x over previous
import functools

import jax
import jax.numpy as jnp
from jax import lax
from jax.experimental import pallas as pl
from jax.experimental.pallas import tpu as pltpu
from jax.experimental.pallas import tpu_sc as plsc

N_NODES = 10000
N_EDGES = 320000
D = 128

NC = 2
NS = 16
NW = NC * NS

CHUNK = 128
G_CHUNKS = (2 * N_EDGES) // CHUNK
S_CHUNKS = N_EDGES // CHUNK
ZROW = 80
ZCHUNKS = N_NODES // ZROW

_mesh = plsc.VectorSubcoreMesh(core_axis_name="c", subcore_axis_name="s")


def _relu(v):
    return jnp.maximum(v, 0.0)


def _dot(a, b):
    return jnp.dot(a, b, preferred_element_type=jnp.float32)


NB = 2
G_GROUPS = (-(-G_CHUNKS // NW) + NB - 1) // NB
S_GROUPS = (-(-S_CHUNKS // NW) + NB - 1) // NB


def _gather_body(table, idx, out, idx_v, buf, si0, si1, sg0, sg1, sw0, sw1):
    c = lax.axis_index("c")
    s = lax.axis_index("s")
    wid = s * NC + c
    si = (si0, si1)
    sg = (sg0, sg1)
    sw = (sw0, sw1)

    for b in range(NB):
        k0 = wid + b * NW

        @pl.when(k0 < G_CHUNKS)
        def _():
            pltpu.async_copy(idx.at[k0], idx_v.at[b], si[b])

    @pl.loop(0, G_GROUPS)
    def _(g):
        for b in range(NB):
            k = wid + (g * NB + b) * NW

            @pl.when(k < G_CHUNKS)
            def _():
                @pl.when(g > 0)
                def _():
                    pltpu.make_async_copy(
                        buf.at[b], out.at[pl.ds(0, CHUNK)], sw[b]).wait()

                pltpu.make_async_copy(idx.at[0], idx_v.at[b], si[b]).wait()
                pltpu.async_copy(table.at[idx_v.at[b]], buf.at[b], sg[b])

        for b in range(NB):
            k = wid + (g * NB + b) * NW

            @pl.when(k < G_CHUNKS)
            def _():
                pltpu.make_async_copy(
                    table.at[idx_v.at[b]], buf.at[b], sg[b]).wait()
                kn = k + NB * NW

                @pl.when(kn < G_CHUNKS)
                def _():
                    pltpu.async_copy(idx.at[kn], idx_v.at[b], si[b])

                pltpu.async_copy(buf.at[b], out.at[pl.ds(k * CHUNK, CHUNK)],
                                 sw[b])

    for b in range(NB):
        k0 = wid + b * NW

        @pl.when(k0 < G_CHUNKS)
        def _():
            pltpu.make_async_copy(buf.at[b], out.at[pl.ds(0, CHUNK)],
                                  sw[b]).wait()


_sc_gather = pl.kernel(
    _gather_body,
    out_type=jax.ShapeDtypeStruct((2 * N_EDGES, D), jnp.float32),
    mesh=_mesh,
    scratch_types=[
        pltpu.VMEM((NB, CHUNK), jnp.int32),
        pltpu.VMEM((NB, CHUNK, D), jnp.float32),
    ] + [pltpu.SemaphoreType.DMA] * 6,
)


def _scatter_body(msg, idx, zeros, out0, out1, shared, idx_v, mbuf,
                  si0, si1, sm0, sm1, ss0, ss1):
    c = lax.axis_index("c")
    s = lax.axis_index("s")
    wid = s * NC + c
    si = (si0, si1)
    sm = (sm0, sm1)
    ss = (ss0, ss1)

    for b in range(NB):
        k0 = wid + b * NW

        @pl.when(k0 < S_CHUNKS)
        def _():
            pltpu.async_copy(idx.at[k0], idx_v.at[b], si[b])
            pltpu.async_copy(msg.at[pl.ds(k0 * CHUNK, CHUNK)], mbuf.at[b],
                             sm[b])

    @pl.loop(s, ZCHUNKS, step=NS)
    def _(z):
        pltpu.sync_copy(zeros.at[pl.ds(z * ZROW, ZROW)],
                        shared.at[pl.ds(z * ZROW, ZROW)])
    plsc.subcore_barrier()

    @pl.loop(0, S_GROUPS)
    def _(g):
        for b in range(NB):
            k = wid + (g * NB + b) * NW

            @pl.when(k < S_CHUNKS)
            def _():
                pltpu.make_async_copy(idx.at[0], idx_v.at[b], si[b]).wait()
                pltpu.make_async_copy(msg.at[pl.ds(0, CHUNK)], mbuf.at[b],
                                      sm[b]).wait()
                pltpu.async_copy(mbuf.at[b], shared.at[idx_v.at[b]], ss[b],
                                 add=True)

        for b in range(NB):
            k = wid + (g * NB + b) * NW

            @pl.when(k < S_CHUNKS)
            def _():
                pltpu.make_async_copy(mbuf.at[b], shared.at[idx_v.at[b]],
                                      ss[b]).wait()
                kn = k + NB * NW

                @pl.when(kn < S_CHUNKS)
                def _():
                    pltpu.async_copy(idx.at[kn], idx_v.at[b], si[b])
                    pltpu.async_copy(msg.at[pl.ds(kn * CHUNK, CHUNK)],
                                     mbuf.at[b], sm[b])

    plsc.subcore_barrier()

    @pl.when(c == 0)
    def _():
        @pl.loop(s, ZCHUNKS, step=NS)
        def _(z):
            pltpu.sync_copy(shared.at[pl.ds(z * ZROW, ZROW)],
                            out0.at[pl.ds(z * ZROW, ZROW)])

    @pl.when(c == 1)
    def _():
        @pl.loop(s, ZCHUNKS, step=NS)
        def _(z):
            pltpu.sync_copy(shared.at[pl.ds(z * ZROW, ZROW)],
                            out1.at[pl.ds(z * ZROW, ZROW)])


_sc_scatter = pl.kernel(
    _scatter_body,
    out_type=(
        jax.ShapeDtypeStruct((N_NODES, D), jnp.float32),
        jax.ShapeDtypeStruct((N_NODES, D), jnp.float32),
    ),
    mesh=_mesh,
    scratch_types=[
        pltpu.VMEM_SHARED((N_NODES, D), jnp.float32),
        pltpu.VMEM((NB, CHUNK), jnp.int32),
        pltpu.VMEM((NB, CHUNK, D), jnp.float32),
    ] + [pltpu.SemaphoreType.DMA] * 6,
)


N_BLK = 1000


def _node_body(x, w0, b0, w1, b1, w2, b2, o):
    h = _relu(_dot(x[...], w0[...]) + b0[...])
    h = _relu(_dot(h, w1[...]) + b1[...])
    o[...] = _dot(h, w2[...]) + b2[...]


BLK_E = 2000
NBLK_E = N_EDGES // BLK_E


def _full(shape):
    return pl.BlockSpec(shape, lambda c: (0, 0))


def _step1_body(ea, xj, xi, e0w, e0b, e1w, e1b, e2w, e2b, e3w, e3b,
                m0w, m0b, m1w, m1b, n0w, n0b, ef_o, msg_o):
    h = _relu(_dot(ea[...], e0w[...]) + e0b[...])
    h = _relu(_dot(h, e1w[...]) + e1b[...])
    h = _relu(_dot(h, e2w[...]) + e2b[...])
    ef = _dot(h, e3w[...]) + e3b[...]
    xiv = xi[...]
    cat = jnp.concatenate([xiv, xj[...]], axis=1)
    m0 = m0w[...]
    h = _relu(_dot(cat, m0[:256]) + _dot(ef, m0[256:]) + m0b[...])
    ef1 = _relu(_dot(h, m1w[...]) + m1b[...])
    ef_o[...] = ef1
    n0 = n0w[...]
    msg_o[...] = _relu(_dot(xiv, n0[:128]) + _dot(ef1, n0[128:]) + n0b[...])


def _step2_body(ef, xj, xi, m0w, m0b, m1w, m1b, n0w, n0b, ef_o, msg_o):
    xiv = xi[...]
    cat = jnp.concatenate([xiv, xj[...]], axis=1)
    m0 = m0w[...]
    h = _relu(_dot(cat, m0[:256]) + _dot(ef[...], m0[256:]) + m0b[...])
    ef1 = _relu(_dot(h, m1w[...]) + m1b[...])
    ef_o[...] = ef1
    n0 = n0w[...]
    msg_o[...] = _relu(_dot(xiv, n0[:128]) + _dot(ef1, n0[128:]) + n0b[...])


def _step3_body(ef, xj, xi, m0w, m0b, m1w, m1b, c0w, c0b, c1w, c1b,
                c2w, c2b, o):
    cat = jnp.concatenate([xi[...], xj[...]], axis=1)
    m0 = m0w[...]
    h = _relu(_dot(cat, m0[:256]) + _dot(ef[...], m0[256:]) + m0b[...])
    ef1 = _relu(_dot(h, m1w[...]) + m1b[...])
    h = _relu(_dot(ef1, c0w[...]) + c0b[...])
    h = _relu(_dot(h, c1w[...]) + c1b[...])
    o[...] = _dot(h, c2w[...]) + c2b[...]


def _combine_body(a, b, o):
    o[...] = a[...] + b[...]


def _edge_spec(width):
    return pl.BlockSpec((BLK_E, width), lambda c: (c, 0))


def _xj_spec():
    return pl.BlockSpec((BLK_E, D), lambda c: (c, 0))


def _xi_spec():
    return pl.BlockSpec((BLK_E, D), lambda c: (c + NBLK_E, 0))


def kernel(x, edge_attr, edge_index, params):
    p = params

    def wb(name):
        w = p[name + "_W"]
        b = p[name + "_b"].reshape(1, -1)
        return w, b

    ne0w, ne0b = wb("ne0"); ne1w, ne1b = wb("ne1"); ne2w, ne2b = wb("ne2")
    ee0w, ee0b = wb("ee0"); ee1w, ee1b = wb("ee1")
    ee2w, ee2b = wb("ee2"); ee3w, ee3b = wb("ee3")
    me0w, me0b = wb("me0"); me1w, me1b = wb("me1")
    mn0w, mn0b = wb("mn0")
    c0w, c0b = wb("c0"); c1w, c1b = wb("c1"); c2w, c2b = wb("c2")

    idx_all = edge_index.reshape(G_CHUNKS, CHUNK)
    idx_i = edge_index[1].reshape(S_CHUNKS, CHUNK)
    zeros = jnp.zeros((N_NODES, D), jnp.float32)

    nf = pl.pallas_call(
        _node_body,
        grid=(N_NODES // N_BLK,),
        in_specs=[
            pl.BlockSpec((N_BLK, D), lambda c: (c, 0)),
            _full((D, D)), _full((1, D)),
            _full((D, 64)), _full((1, 64)),
            _full((64, D)), _full((1, D)),
        ],
        out_specs=pl.BlockSpec((N_BLK, D), lambda c: (c, 0)),
        out_shape=jax.ShapeDtypeStruct((N_NODES, D), jnp.float32),
    )(x, ne0w, ne0b, ne1w, ne1b, ne2w, ne2b)

    g = _sc_gather(nf, idx_all)
    ef, msg = pl.pallas_call(
        _step1_body,
        grid=(NBLK_E,),
        in_specs=[
            _edge_spec(16), _xj_spec(), _xi_spec(),
            _full((16, 32)), _full((1, 32)),
            _full((32, 64)), _full((1, 64)),
            _full((64, 64)), _full((1, 64)),
            _full((64, 16)), _full((1, 16)),
            _full((272, 64)), _full((1, 64)),
            _full((64, 16)), _full((1, 16)),
            _full((144, D)), _full((1, D)),
        ],
        out_specs=[_edge_spec(16), _edge_spec(D)],
        out_shape=[
            jax.ShapeDtypeStruct((N_EDGES, 16), jnp.float32),
            jax.ShapeDtypeStruct((N_EDGES, D), jnp.float32),
        ],
    )(edge_attr, g, g, ee0w, ee0b, ee1w, ee1b, ee2w, ee2b, ee3w, ee3b,
      me0w, me0b, me1w, me1b, mn0w, mn0b)

    def _segment_sum(m):
        p0, p1 = _sc_scatter(m, idx_i, zeros)
        return pl.pallas_call(
            _combine_body,
            grid=(N_NODES // N_BLK,),
            in_specs=[pl.BlockSpec((N_BLK, D), lambda c: (c, 0))] * 2,
            out_specs=pl.BlockSpec((N_BLK, D), lambda c: (c, 0)),
            out_shape=jax.ShapeDtypeStruct((N_NODES, D), jnp.float32),
        )(p0, p1)

    nf = _segment_sum(msg)

    g = _sc_gather(nf, idx_all)
    ef, msg = pl.pallas_call(
        _step2_body,
        grid=(NBLK_E,),
        in_specs=[
            _edge_spec(16), _xj_spec(), _xi_spec(),
            _full((272, 64)), _full((1, 64)),
            _full((64, 16)), _full((1, 16)),
            _full((144, D)), _full((1, D)),
        ],
        out_specs=[_edge_spec(16), _edge_spec(D)],
        out_shape=[
            jax.ShapeDtypeStruct((N_EDGES, 16), jnp.float32),
            jax.ShapeDtypeStruct((N_EDGES, D), jnp.float32),
        ],
    )(ef, g, g, me0w, me0b, me1w, me1b, mn0w, mn0b)
    nf = _segment_sum(msg)

    g = _sc_gather(nf, idx_all)
    out = pl.pallas_call(
        _step3_body,
        grid=(NBLK_E,),
        in_specs=[
            _edge_spec(16), _xj_spec(), _xi_spec(),
            _full((272, 64)), _full((1, 64)),
            _full((64, 16)), _full((1, 16)),
            _full((16, 64)), _full((1, 64)),
            _full((64, 32)), _full((1, 32)),
            _full((32, 1)), _full((1, 1)),
        ],
        out_specs=_edge_spec(1),
        out_shape=jax.ShapeDtypeStruct((N_EDGES, 1), jnp.float32),
    )(ef, g, g, me0w, me0b, me1w, me1b, c0w, c0b, c1w, c1b, c2w, c2b)
    return out
